# baseline (speedup 1.0000x reference)
import functools

import jax
import jax.numpy as jnp
from jax import lax
from jax.experimental import pallas as pl
from jax.experimental.pallas import tpu as pltpu
from jax.experimental.pallas import tpu_sc as plsc

B = 16384
DIM = 32
NC = 2
NS = 16
NW = NC * NS
CHUNK = 128
CPW = B // NW // CHUNK


def _sc_gather_body(uids, iids, user_table, item_table, out_u, out_i,
                    idx_u, idx_i, rows_u, rows_i, sem_u, sem_i):
    wid = lax.axis_index("s") * NC + lax.axis_index("c")
    base = wid * CPW
    pltpu.sync_copy(uids.at[pl.ds(base, CPW)], idx_u)
    pltpu.sync_copy(iids.at[pl.ds(base, CPW)], idx_i)
    copies = []
    for j in range(CPW):
        copies.append(pltpu.async_copy(user_table.at[idx_u.at[j]], rows_u.at[j], sem_u))
        copies.append(pltpu.async_copy(item_table.at[idx_i.at[j]], rows_i.at[j], sem_i))
    for c in copies:
        c.wait()
    pltpu.sync_copy(rows_u, out_u.at[pl.ds(base, CPW)])
    pltpu.sync_copy(rows_i, out_i.at[pl.ds(base, CPW)])


_sc_gather = functools.partial(
    pl.kernel,
    out_type=(
        jax.ShapeDtypeStruct((B // CHUNK, CHUNK, DIM), jnp.float32),
        jax.ShapeDtypeStruct((B // CHUNK, CHUNK, DIM), jnp.float32),
    ),
    mesh=plsc.VectorSubcoreMesh(core_axis_name="c", subcore_axis_name="s"),
    scratch_types=[
        pltpu.VMEM((CPW, CHUNK), jnp.int32),
        pltpu.VMEM((CPW, CHUNK), jnp.int32),
        pltpu.VMEM((CPW, CHUNK, DIM), jnp.float32),
        pltpu.VMEM((CPW, CHUNK, DIM), jnp.float32),
        pltpu.SemaphoreType.DMA,
        pltpu.SemaphoreType.DMA,
    ],
    compiler_params=pltpu.CompilerParams(use_tc_tiling_on_sc=False),
)(_sc_gather_body)


def _tc_dense_body(ue_ref, ie_ref, wu_ref, bu_ref, wi_ref, bi_ref, out_ref):
    u = jnp.maximum(
        jnp.dot(ue_ref[...], wu_ref[...], preferred_element_type=jnp.float32)
        + bu_ref[...], 0.0)
    v = jnp.maximum(
        jnp.dot(ie_ref[...], wi_ref[...], preferred_element_type=jnp.float32)
        + bi_ref[...], 0.0)
    out_ref[...] = jnp.sum(u * v, axis=1, keepdims=True)


def _tc_dense(ue, ie, Wu, bu2, Wi, bi2):
    blk = 2048
    grid = B // blk
    return pl.pallas_call(
        _tc_dense_body,
        grid=(grid,),
        in_specs=[
            pl.BlockSpec((blk, DIM), lambda i: (i, 0)),
            pl.BlockSpec((blk, DIM), lambda i: (i, 0)),
            pl.BlockSpec((DIM, DIM), lambda i: (0, 0)),
            pl.BlockSpec((1, DIM), lambda i: (0, 0)),
            pl.BlockSpec((DIM, DIM), lambda i: (0, 0)),
            pl.BlockSpec((1, DIM), lambda i: (0, 0)),
        ],
        out_specs=pl.BlockSpec((blk, 1), lambda i: (i, 0)),
        out_shape=jax.ShapeDtypeStruct((B, 1), jnp.float32),
    )(ue, ie, Wu, bu2, Wi, bi2)


def kernel(user_table, item_table, Wu, bu, Wi, bi, user_ids, item_ids):
    uids = user_ids.reshape(B // CHUNK, CHUNK)
    iids = item_ids.reshape(B // CHUNK, CHUNK)
    ue3, ie3 = _sc_gather(uids, iids, user_table, item_table)
    ue = ue3.reshape(B, DIM)
    ie = ie3.reshape(B, DIM)
    scores = _tc_dense(ue, ie, Wu, bu.reshape(1, DIM), Wi, bi.reshape(1, DIM))
    return scores.reshape(B)
